# Initial kernel scaffold; baseline (speedup 1.0000x reference)
import jax
import jax.numpy as jnp
from jax.experimental import pallas as pl


def kernel(q_sub, q_rel, hidden, path_state, edges, nodes, old_nodes_new_idx, batchsize, rela_embed, Ws_attn, Wr_attn, Wqr_attn_w, Wqr_attn_b, w_alpha_w, w_alpha_b, W_h, W_path_prev, W_path_rel, curvature):
    raise NotImplementedError("write your pallas kernel here")


# speedup vs baseline: 1.4121x; 1.4121x over previous
import functools
import numpy as np
import jax
import jax.numpy as jnp
from jax import lax
from jax.experimental import pallas as pl
from jax.experimental.pallas import tpu as pltpu
from jax.experimental.pallas import tpu_sc as plsc

_MIN = 1e-15
_NPAD = 10240
_NHALF = _NPAD // 2
_TRASH = _NHALF
_NEDGE = 320000
_CHUNK = 128
_SPT = 20
_CPW = _SPT * 8
_EPAD = 16 * _CPW * _CHUNK
_NSC = _EPAD // (8 * _CHUNK)
_NROW = _EPAD // _CHUNK
_ROWS_PER_TILE = _NHALF // 16


def _expmap0_rows(u, sqrtc):
    un = jnp.maximum(jnp.sqrt(jnp.sum(u * u, axis=-1, keepdims=True)), _MIN)
    g = jnp.tanh(jnp.clip(sqrtc * un, -15.0, 15.0)) * u / (sqrtc * un)
    gn = jnp.maximum(jnp.sqrt(jnp.sum(g * g, axis=-1, keepdims=True)), _MIN)
    mx = (1.0 - 0.004) / sqrtc
    return jnp.where(gn > mx, g / gn * mx, g)


def _pre_body(cv, h, p, r, ws, wr, wq, bq, wpp, wpr,
              zs, eh, pp, zr, zq, er, pr):
    sqrtc = jnp.sqrt(jnp.maximum(cv[0, 0], 1e-6))
    hv = h[...]
    rv = r[...]
    zs[...] = jnp.dot(hv, ws[...], preferred_element_type=jnp.float32)
    zr[...] = jnp.dot(rv, wr[...], preferred_element_type=jnp.float32)
    zq[...] = jnp.dot(rv, wq[...], preferred_element_type=jnp.float32) + bq[...]
    eh[...] = _expmap0_rows(hv, sqrtc)
    er[...] = _expmap0_rows(rv, sqrtc)
    pp[...] = jnp.dot(p[...], wpp[...], preferred_element_type=jnp.float32)
    pr[...] = jnp.dot(rv, wpr[...], preferred_element_type=jnp.float32)


def _post_body(cv, m, wh, hout):
    sqrtc = jnp.sqrt(jnp.maximum(cv[0, 0], 1e-6))
    a = jnp.dot(m[...], wh[...], preferred_element_type=jnp.float32)
    a = _expmap0_rows(a, sqrtc)
    an = jnp.maximum(jnp.sqrt(jnp.sum(a * a, axis=-1, keepdims=True)), _MIN)
    t = jnp.clip(sqrtc * an, -1.0 + 1e-05, 1.0 - 1e-05)
    art = 0.5 * (jnp.log1p(t) - jnp.log1p(-t))
    hout[...] = a / an / sqrtc * art


def _vsqrt(n2):
    i = plsc.bitcast(n2, jnp.int32)
    y = plsc.bitcast(np.int32(0x5F3759DF) - (i >> 1), jnp.float32)
    for _ in range(3):
        y = y * (1.5 - 0.5 * n2 * y * y)
    return n2 * y


def _vln(q):
    bits = plsc.bitcast(q, jnp.int32)
    e = ((bits >> 23) & 0xFF) - 127
    m = plsc.bitcast((bits & 0x7FFFFF) | np.int32(0x3F800000), jnp.float32)
    r = (m - 1.0) / (m + 1.0)
    r2 = r * r
    lnm = 2.0 * r * (1.0 + r2 * (1.0 / 3.0 + r2 * (0.2 + r2 * (1.0 / 7.0 + r2 / 9.0))))
    return e.astype(jnp.float32) * np.float32(np.log(2.0)) + lnm


def _bf16_pair(lo, hi):
    bl = plsc.bitcast(lo, jnp.int32)
    bh = plsc.bitcast(hi, jnp.int32)
    rl = lax.shift_right_logical(bl + 0x7FFF + ((bl >> 16) & 1), 16) & 0xFFFF
    rh = lax.shift_right_logical(bh + 0x7FFF + ((bh >> 16) & 1), 16) & 0xFFFF
    return rl | (rh << 16)


def _zero_acc(sid, buf128, buf64, acc, zero16):
    r0 = sid * _ROWS_PER_TILE
    pltpu.sync_copy(buf128, acc.at[pl.ds(r0, 128)])
    pltpu.sync_copy(buf128, acc.at[pl.ds(r0 + 128, 128)])
    pltpu.sync_copy(buf64, acc.at[pl.ds(r0 + 256, 64)])


def _localize_obj(ov, cid, trash):
    loc = ov - cid * _NHALF
    ok = jnp.logical_and(loc >= 0, loc < _NHALF)
    return jnp.where(ok, loc, trash)


def _sc_msg_kernel(ix_h, qrel_h, zs_h, zr_h, zq_h, eh_h, er_h,
                   omsg, oalpha,
                   qrel_t, ixb, qidxb, objl, afb, pka,
                   zb, ehb, erb,
                   msg_acc,
                   s_z, s_eh, s_er):
    cid = lax.axis_index("c")
    sid = lax.axis_index("s")

    pltpu.sync_copy(qrel_h, qrel_t)

    zero16 = jnp.zeros((16,), jnp.float32)

    def _zrow(i, _):
        for k in range(8):
            zb[i, pl.ds(16 * k, 16)] = zero16
        return 0
    lax.fori_loop(0, _CHUNK, _zrow, 0)
    _zero_acc(sid, zb, zb.at[pl.ds(0, 64)], msg_acc, zero16)
    plsc.subcore_barrier()

    lane = lax.iota(jnp.int32, 16)
    trash = jnp.full((16,), _TRASH, jnp.int32)

    def _chunk(ci, _):
        sc = sid * _SPT + (ci >> 3)
        j = jnp.bitwise_and(ci, 7)
        rowi = sc * 8 + j
        be = rowi * _CHUNK

        @pl.when(j == 0)
        def _ld_ix():
            pltpu.sync_copy(ix_h.at[pl.ds(sc * 32, 32)], ixb)

        subr = ixb.at[j]
        relr = ixb.at[8 + j]
        g_z = pltpu.async_copy(zs_h.at[subr], zb, s_z)
        g_eh = pltpu.async_copy(eh_h.at[subr], ehb, s_eh)
        g_er = pltpu.async_copy(er_h.at[relr], erb, s_er)

        def _qg(g, _):
            sl = pl.ds(16 * g, 16)
            rv = ixb[16 + j, sl]
            hi = lax.shift_right_logical(rv, 7)
            lo = jnp.bitwise_and(rv, 127)
            qidxb[0, sl] = plsc.load_gather(qrel_t, [hi, lo])
            objl[0, sl] = _localize_obj(ixb[24 + j, sl], cid, trash)
            return 0
        lax.fori_loop(0, _CHUNK // 16, _qg, 0)

        g_z.wait()
        g_z2 = pltpu.async_copy(zr_h.at[relr], zb, s_z, add=True)
        g_z2.wait()
        g_z3 = pltpu.async_copy(zq_h.at[qidxb.at[0]], zb, s_z, add=True)
        g_z3.wait(); g_eh.wait(); g_er.wait()

        cc = plsc.bitcast(qrel_t[81, pl.ds(0, 16)], jnp.float32)
        scv = plsc.bitcast(qrel_t[81, pl.ds(16, 16)], jnp.float32)
        mxv = plsc.bitcast(qrel_t[81, pl.ds(32, 16)], jnp.float32)
        wbv = plsc.bitcast(qrel_t[81, pl.ds(48, 16)], jnp.float32)
        wvecs = [plsc.bitcast(qrel_t[80, pl.ds(16 * k, 16)], jnp.float32)
                 for k in range(8)]

        def _group(g, _):
            base = g * 16
            zdv = jnp.zeros((16,), jnp.float32)
            xyv = jnp.zeros((16,), jnp.float32)
            x2v = jnp.zeros((16,), jnp.float32)
            y2v = jnp.zeros((16,), jnp.float32)
            for e in range(16):
                row = base + e
                az = axy = ax2 = ay2 = None
                for k in range(8):
                    sl = pl.ds(16 * k, 16)
                    zk = jnp.maximum(zb[row, sl], 0.0) * wvecs[k]
                    xk = ehb[row, sl]
                    yk = erb[row, sl]
                    if k == 0:
                        az, axy, ax2, ay2 = zk, xk * yk, xk * xk, yk * yk
                    else:
                        az = az + zk
                        axy = axy + xk * yk
                        ax2 = ax2 + xk * xk
                        ay2 = ay2 + yk * yk
                msk = lane == e
                zdv = jnp.where(msk, jnp.sum(az), zdv)
                xyv = jnp.where(msk, jnp.sum(axy), xyv)
                x2v = jnp.where(msk, jnp.sum(ax2), x2v)
                y2v = jnp.where(msk, jnp.sum(ay2), y2v)

            alpha = 1.0 / (1.0 + jnp.exp(-(zdv + wbv)))
            den = jnp.maximum(1.0 + 2.0 * cc * xyv + cc * cc * x2v * y2v, _MIN)
            av = (1.0 + 2.0 * cc * xyv + cc * y2v) / den
            bv = (1.0 - cc * x2v) / den
            n2 = jnp.maximum(av * av * x2v + 2.0 * av * bv * xyv + bv * bv * y2v, 0.0)
            n = _vsqrt(n2)
            ncl = jnp.maximum(n, _MIN)
            scale1 = jnp.where(ncl > mxv, mxv / ncl, 1.0)
            yn = jnp.maximum(n * scale1, _MIN)
            t = scv * yn
            art = 0.5 * _vln((1.0 + t) / (1.0 - t))
            kms = scale1 * art / t
            eidv = lane + (be + base)
            vf = jnp.where(eidv < _NEDGE, 1.0, 0.0)
            akm = alpha * kms * vf
            w1v = akm * av
            w2v = akm * bv
            afb[0, pl.ds(base, 16)] = alpha * vf

            for e in range(16):
                row = base + e
                w1e = w1v[e]
                w2e = w2v[e]
                for k in range(8):
                    sl = pl.ds(16 * k, 16)
                    zb[row, sl] = w1e * ehb[row, sl] + w2e * erb[row, sl]
            return 0

        lax.fori_loop(0, _CHUNK // 16, _group, 0)

        for g in range(4):
            sl = pl.ds(16 * g, 16)
            pka[0, sl] = _bf16_pair(afb[0, sl], afb[0, pl.ds(64 + 16 * g, 16)])
        pltpu.sync_copy(pka, oalpha.at[cid, pl.ds(rowi, 1)])

        pltpu.sync_copy(zb, msg_acc.at[objl.at[0]], add=True)
        return 0

    lax.fori_loop(0, _CPW, _chunk, 0)
    plsc.subcore_barrier()

    r0 = sid * _ROWS_PER_TILE
    pltpu.sync_copy(msg_acc.at[pl.ds(r0, _ROWS_PER_TILE)],
                    omsg.at[cid, pl.ds(r0, _ROWS_PER_TILE)])


def _sc_path_kernel(ix_h, al_h, pp_h, pr_h,
                    opath,
                    ixb, alb, afb, objl, ppb, pb,
                    path_acc,
                    s_pp):
    cid = lax.axis_index("c")
    sid = lax.axis_index("s")

    zero16 = jnp.zeros((16,), jnp.float32)

    def _zrow(i, _):
        for k in range(8):
            pb[i, pl.ds(16 * k, 16)] = zero16
        return 0
    lax.fori_loop(0, _CHUNK, _zrow, 0)
    _zero_acc(sid, pb, pb.at[pl.ds(0, 64)], path_acc, zero16)
    plsc.subcore_barrier()

    lane = lax.iota(jnp.int32, 16)
    trash = jnp.full((16,), _TRASH, jnp.int32)

    def _chunk(ci, _):
        sc = sid * _SPT + (ci >> 3)
        j = jnp.bitwise_and(ci, 7)
        rowi = sc * 8 + j

        @pl.when(j == 0)
        def _ld_ix():
            pltpu.sync_copy(ix_h.at[pl.ds(sc * 32, 32)], ixb)

        subr = ixb.at[j]
        relr = ixb.at[8 + j]
        g_pp = pltpu.async_copy(pp_h.at[subr], ppb, s_pp)
        c_al = pltpu.async_copy(al_h.at[pl.ds(rowi, 1)], alb, s_pp)

        def _og(g, _):
            sl = pl.ds(16 * g, 16)
            objl[0, sl] = _localize_obj(ixb[24 + j, sl], cid, trash)
            return 0
        lax.fori_loop(0, _CHUNK // 16, _og, 0)

        g_pp.wait()
        c_al.wait()
        g_pp2 = pltpu.async_copy(pr_h.at[relr], ppb, s_pp, add=True)

        for g2 in range(4):
            w = alb[0, pl.ds(16 * g2, 16)]
            afb[0, pl.ds(16 * g2, 16)] = plsc.bitcast(w << 16, jnp.float32)
            afb[0, pl.ds(64 + 16 * g2, 16)] = plsc.bitcast(
                jnp.bitwise_and(w, jnp.int32(-65536)), jnp.float32)
        g_pp2.wait()

        def _group(g, _):
            base = g * 16
            avs = afb[0, pl.ds(base, 16)]
            for e in range(16):
                row = base + e
                ae = avs[e]
                for k in range(4):
                    sl = pl.ds(16 * k, 16)
                    v = ppb[row, sl]
                    th = 1.0 - 2.0 / (jnp.exp(2.0 * v) + 1.0)
                    pb[row, sl] = ae * th
            return 0

        lax.fori_loop(0, _CHUNK // 16, _group, 0)

        pltpu.sync_copy(pb, path_acc.at[objl.at[0]], add=True)
        return 0

    lax.fori_loop(0, _CPW, _chunk, 0)
    plsc.subcore_barrier()

    r0 = sid * _ROWS_PER_TILE
    pltpu.sync_copy(path_acc.at[pl.ds(r0, _ROWS_PER_TILE)],
                    opath.at[cid, pl.ds(r0, _ROWS_PER_TILE)])


def kernel(q_sub, q_rel, hidden, path_state, edges, nodes, old_nodes_new_idx,
           batchsize, rela_embed, Ws_attn, Wr_attn, Wqr_attn_w, Wqr_attn_b,
           w_alpha_w, w_alpha_b, W_h, W_path_prev, W_path_rel, curvature):
    f32 = jnp.float32
    i32 = jnp.int32
    n_node = hidden.shape[0]
    c = jnp.maximum(curvature.astype(f32), 1e-6)
    sqrtc = jnp.sqrt(c)
    mxn = (1.0 - 0.004) / sqrtc
    cvec = jnp.zeros((1, 128), f32).at[0, 0].set(c)

    hid_p = jnp.pad(hidden, ((0, _NPAD - hidden.shape[0]), (0, 0)))
    pth_p = jnp.pad(path_state, ((0, _NPAD - path_state.shape[0]), (0, 0)))
    rel_p = jnp.pad(rela_embed, ((0, _NPAD - rela_embed.shape[0]), (0, 0)))
    bq = jnp.broadcast_to(Wqr_attn_b[None, :], (1, 128))

    nblk = _NPAD // 256
    row_spec = lambda d: pl.BlockSpec((256, d), lambda i: (i, 0))
    full = lambda a, b: pl.BlockSpec((a, b), lambda i: (0, 0))
    out128 = jax.ShapeDtypeStruct((_NPAD, 128), f32)
    zs, eh, pp, zr, zq, er, pr = pl.pallas_call(
        _pre_body,
        grid=(nblk,),
        in_specs=[full(1, 128), row_spec(128), row_spec(64), row_spec(128),
                  full(128, 128), full(128, 128), full(128, 128), full(1, 128),
                  full(64, 128), full(128, 128)],
        out_specs=[row_spec(128)] * 7,
        out_shape=[out128] * 7,
    )(cvec, hid_p, pth_p, rel_p, Ws_attn, Wr_attn, Wqr_attn_w, bq,
      jnp.pad(W_path_prev, ((0, 0), (0, 64))),
      jnp.pad(W_path_rel, ((0, 0), (0, 64))))

    epad = _EPAD - edges.shape[0]
    col = lambda j: jnp.pad(edges[:, j].astype(i32), (0, epad)).reshape(_NSC, 8, _CHUNK)
    ix = jnp.stack([col(4), col(2), col(0), col(5)], axis=1)
    ix = ix.reshape(_NSC * 32, _CHUNK)
    qrel2d = jnp.pad(q_rel.astype(i32), (0, _NPAD - q_rel.shape[0])).reshape(80, 128)
    wal_row = lax.bitcast_convert_type(w_alpha_w[:, 0], i32)[None, :]
    cst_row = lax.bitcast_convert_type(
        jnp.repeat(jnp.stack([c, sqrtc, mxn, w_alpha_b[0]]), 16), i32)
    cst_row = jnp.pad(cst_row, (0, 64))[None, :]
    qrel_aux = jnp.concatenate(
        [qrel2d, wal_row, cst_row, jnp.zeros((6, 128), i32)], axis=0)

    mesh = plsc.VectorSubcoreMesh(core_axis_name="c", subcore_axis_name="s")
    cpar = pltpu.CompilerParams(needs_layout_passes=False)
    msg_call = pl.kernel(
        _sc_msg_kernel,
        out_type=[jax.ShapeDtypeStruct((2, _NHALF, 128), f32),
                  jax.ShapeDtypeStruct((2, _NROW, 64), i32)],
        mesh=mesh,
        compiler_params=cpar,
        scratch_types=[
            pltpu.VMEM((88, 128), i32),
            pltpu.VMEM((32, 128), i32),
            pltpu.VMEM((1, _CHUNK), i32),
            pltpu.VMEM((1, _CHUNK), i32),
            pltpu.VMEM((1, _CHUNK), f32),
            pltpu.VMEM((1, 64), i32),
            pltpu.VMEM((_CHUNK, 128), f32),
            pltpu.VMEM((_CHUNK, 128), f32),
            pltpu.VMEM((_CHUNK, 128), f32),
            pltpu.VMEM_SHARED((_NHALF + 8, 128), f32),
            pltpu.SemaphoreType.DMA,
            pltpu.SemaphoreType.DMA,
            pltpu.SemaphoreType.DMA,
        ],
    )
    omsg, oalpha = msg_call(ix, qrel_aux, zs, zr, zq, eh, er)

    path_call = pl.kernel(
        _sc_path_kernel,
        out_type=[jax.ShapeDtypeStruct((2, _NHALF, 128), f32)],
        mesh=mesh,
        compiler_params=cpar,
        scratch_types=[
            pltpu.VMEM((32, 128), i32),
            pltpu.VMEM((1, 64), i32),
            pltpu.VMEM((1, _CHUNK), f32),
            pltpu.VMEM((1, _CHUNK), i32),
            pltpu.VMEM((_CHUNK, 128), f32),
            pltpu.VMEM((_CHUNK, 128), f32),
            pltpu.VMEM_SHARED((_NHALF + 8, 128), f32),
            pltpu.SemaphoreType.DMA,
        ],
    )
    opath, = path_call(ix, oalpha[0], pp, pr)

    magg = omsg.reshape(_NPAD, 128)
    path_new = opath.reshape(_NPAD, 128)[:n_node, :64]

    hval, = pl.pallas_call(
        _post_body,
        grid=(nblk,),
        in_specs=[full(1, 128), row_spec(128), full(128, 128)],
        out_specs=[row_spec(128)],
        out_shape=[out128],
    )(cvec, magg, W_h)

    return hval[:n_node], path_new


# speedup vs baseline: 1.5668x; 1.1095x over previous
import functools
import numpy as np
import jax
import jax.numpy as jnp
from jax import lax
from jax.experimental import pallas as pl
from jax.experimental.pallas import tpu as pltpu
from jax.experimental.pallas import tpu_sc as plsc

_MIN = 1e-15
_NPAD = 10240
_NHALF = _NPAD // 2
_TRASH = _NHALF
_NEDGE = 320000
_CHUNK = 128
_SPT = 20
_CPW = _SPT * 8
_EPAD = 16 * _CPW * _CHUNK
_NSC = _EPAD // (8 * _CHUNK)
_NROW = _EPAD // _CHUNK
_ROWS_PER_TILE = _NHALF // 16


def _expmap0_rows(u, sqrtc):
    un = jnp.maximum(jnp.sqrt(jnp.sum(u * u, axis=-1, keepdims=True)), _MIN)
    g = jnp.tanh(jnp.clip(sqrtc * un, -15.0, 15.0)) * u / (sqrtc * un)
    gn = jnp.maximum(jnp.sqrt(jnp.sum(g * g, axis=-1, keepdims=True)), _MIN)
    mx = (1.0 - 0.004) / sqrtc
    return jnp.where(gn > mx, g / gn * mx, g)


def _pre_body(cv, h, p, r, ws, wr, wq, bq, wpp, wpr,
              zs, eh, pp, zr, zq, er, pr):
    sqrtc = jnp.sqrt(jnp.maximum(cv[0, 0], 1e-6))
    hv = h[...]
    rv = r[...]
    zs[...] = jnp.dot(hv, ws[...], preferred_element_type=jnp.float32)
    zr[...] = jnp.dot(rv, wr[...], preferred_element_type=jnp.float32)
    zq[...] = jnp.dot(rv, wq[...], preferred_element_type=jnp.float32) + bq[...]
    eh[...] = _expmap0_rows(hv, sqrtc)
    er[...] = _expmap0_rows(rv, sqrtc)
    pp[...] = jnp.dot(p[...], wpp[...], preferred_element_type=jnp.float32)
    pr[...] = jnp.dot(rv, wpr[...], preferred_element_type=jnp.float32)


def _post_body(cv, m, wh, hout):
    sqrtc = jnp.sqrt(jnp.maximum(cv[0, 0], 1e-6))
    a = jnp.dot(m[...], wh[...], preferred_element_type=jnp.float32)
    a = _expmap0_rows(a, sqrtc)
    an = jnp.maximum(jnp.sqrt(jnp.sum(a * a, axis=-1, keepdims=True)), _MIN)
    t = jnp.clip(sqrtc * an, -1.0 + 1e-05, 1.0 - 1e-05)
    art = 0.5 * (jnp.log1p(t) - jnp.log1p(-t))
    hout[...] = a / an / sqrtc * art


def _vsqrt(n2):
    i = plsc.bitcast(n2, jnp.int32)
    y = plsc.bitcast(np.int32(0x5F3759DF) - (i >> 1), jnp.float32)
    for _ in range(3):
        y = y * (1.5 - 0.5 * n2 * y * y)
    return n2 * y


def _vln(q):
    bits = plsc.bitcast(q, jnp.int32)
    e = ((bits >> 23) & 0xFF) - 127
    m = plsc.bitcast((bits & 0x7FFFFF) | np.int32(0x3F800000), jnp.float32)
    r = (m - 1.0) / (m + 1.0)
    r2 = r * r
    lnm = 2.0 * r * (1.0 + r2 * (1.0 / 3.0 + r2 * (0.2 + r2 * (1.0 / 7.0 + r2 / 9.0))))
    return e.astype(jnp.float32) * np.float32(np.log(2.0)) + lnm


def _bf16_pair(lo, hi):
    bl = plsc.bitcast(lo, jnp.int32)
    bh = plsc.bitcast(hi, jnp.int32)
    rl = lax.shift_right_logical(bl + 0x7FFF + ((bl >> 16) & 1), 16) & 0xFFFF
    rh = lax.shift_right_logical(bh + 0x7FFF + ((bh >> 16) & 1), 16) & 0xFFFF
    return rl | (rh << 16)


def _zero_acc(sid, buf128, buf64, acc, zero16):
    r0 = sid * _ROWS_PER_TILE
    pltpu.sync_copy(buf128, acc.at[pl.ds(r0, 128)])
    pltpu.sync_copy(buf128, acc.at[pl.ds(r0 + 128, 128)])
    pltpu.sync_copy(buf64, acc.at[pl.ds(r0 + 256, 64)])


def _localize_obj(ov, cid, trash):
    loc = ov - cid * _NHALF
    ok = jnp.logical_and(loc >= 0, loc < _NHALF)
    return jnp.where(ok, loc, trash)


def _sc_msg_kernel(ix_h, qrel_h, zs_h, zr_h, zq_h, eh_h, er_h,
                   omsg, oalpha,
                   qrel_t, ixb, qidxb, objl, afb, pka,
                   zb, ehb, erb,
                   msg_acc,
                   s_z, s_eh, s_er, s_q):
    cid = lax.axis_index("c")
    sid = lax.axis_index("s")

    pltpu.sync_copy(qrel_h, qrel_t)

    zero16 = jnp.zeros((16,), jnp.float32)

    def _zrow(i, _):
        for k in range(8):
            zb[i, pl.ds(16 * k, 16)] = zero16
        return 0
    lax.fori_loop(0, _CHUNK, _zrow, 0)
    _zero_acc(sid, zb, zb.at[pl.ds(0, 64)], msg_acc, zero16)
    plsc.subcore_barrier()

    lane = lax.iota(jnp.int32, 16)
    trash = jnp.full((16,), _TRASH, jnp.int32)

    def _chunk(ci, _):
        sc = sid * _SPT + (ci >> 3)
        j = jnp.bitwise_and(ci, 7)
        rowi = sc * 8 + j
        be = rowi * _CHUNK

        @pl.when(j == 0)
        def _ld_ix():
            pltpu.sync_copy(ix_h.at[pl.ds(sc * 32, 32)], ixb)

        subr = ixb.at[j]
        relr = ixb.at[8 + j]
        g_z = pltpu.async_copy(zs_h.at[subr], zb, s_z)
        g_eh = pltpu.async_copy(eh_h.at[subr], ehb, s_eh)
        g_er = pltpu.async_copy(er_h.at[relr], erb, s_er)

        def _qg(g, _):
            sl = pl.ds(16 * g, 16)
            rv = ixb[16 + j, sl]
            hi = lax.shift_right_logical(rv, 7)
            lo = jnp.bitwise_and(rv, 127)
            qidxb[0, sl] = plsc.load_gather(qrel_t, [hi, lo])
            objl[0, sl] = _localize_obj(ixb[24 + j, sl], cid, trash)
            return 0
        lax.fori_loop(0, _CHUNK // 16, _qg, 0)

        g_z.wait()
        g_z2 = pltpu.async_copy(zr_h.at[relr], zb, s_z, add=True)
        g_z3 = pltpu.async_copy(zq_h.at[qidxb.at[0]], zb, s_q, add=True)
        g_z2.wait(); g_z3.wait(); g_eh.wait(); g_er.wait()

        cc = plsc.bitcast(qrel_t[81, pl.ds(0, 16)], jnp.float32)
        scv = plsc.bitcast(qrel_t[81, pl.ds(16, 16)], jnp.float32)
        mxv = plsc.bitcast(qrel_t[81, pl.ds(32, 16)], jnp.float32)
        wbv = plsc.bitcast(qrel_t[81, pl.ds(48, 16)], jnp.float32)
        wvecs = [plsc.bitcast(qrel_t[80, pl.ds(16 * k, 16)], jnp.float32)
                 for k in range(8)]

        def _group(g, _):
            base = g * 16
            zdv = jnp.zeros((16,), jnp.float32)
            xyv = jnp.zeros((16,), jnp.float32)
            x2v = jnp.zeros((16,), jnp.float32)
            y2v = jnp.zeros((16,), jnp.float32)
            for e in range(16):
                row = base + e
                az = axy = ax2 = ay2 = None
                for k in range(8):
                    sl = pl.ds(16 * k, 16)
                    zk = jnp.maximum(zb[row, sl], 0.0) * wvecs[k]
                    xk = ehb[row, sl]
                    yk = erb[row, sl]
                    if k == 0:
                        az, axy, ax2, ay2 = zk, xk * yk, xk * xk, yk * yk
                    else:
                        az = az + zk
                        axy = axy + xk * yk
                        ax2 = ax2 + xk * xk
                        ay2 = ay2 + yk * yk
                msk = lane == e
                zdv = jnp.where(msk, jnp.sum(az), zdv)
                xyv = jnp.where(msk, jnp.sum(axy), xyv)
                x2v = jnp.where(msk, jnp.sum(ax2), x2v)
                y2v = jnp.where(msk, jnp.sum(ay2), y2v)

            alpha = 1.0 / (1.0 + jnp.exp(-(zdv + wbv)))
            den = jnp.maximum(1.0 + 2.0 * cc * xyv + cc * cc * x2v * y2v, _MIN)
            av = (1.0 + 2.0 * cc * xyv + cc * y2v) / den
            bv = (1.0 - cc * x2v) / den
            n2 = jnp.maximum(av * av * x2v + 2.0 * av * bv * xyv + bv * bv * y2v, 0.0)
            n = _vsqrt(n2)
            ncl = jnp.maximum(n, _MIN)
            scale1 = jnp.where(ncl > mxv, mxv / ncl, 1.0)
            yn = jnp.maximum(n * scale1, _MIN)
            t = scv * yn
            art = 0.5 * _vln((1.0 + t) / (1.0 - t))
            kms = scale1 * art / t
            eidv = lane + (be + base)
            vf = jnp.where(eidv < _NEDGE, 1.0, 0.0)
            akm = alpha * kms * vf
            w1v = akm * av
            w2v = akm * bv
            afb[0, pl.ds(base, 16)] = alpha * vf

            for e in range(16):
                row = base + e
                w1e = w1v[e]
                w2e = w2v[e]
                for k in range(8):
                    sl = pl.ds(16 * k, 16)
                    zb[row, sl] = w1e * ehb[row, sl] + w2e * erb[row, sl]
            return 0

        lax.fori_loop(0, _CHUNK // 16, _group, 0)

        for g in range(4):
            sl = pl.ds(16 * g, 16)
            pka[0, sl] = _bf16_pair(afb[0, sl], afb[0, pl.ds(64 + 16 * g, 16)])
        pltpu.sync_copy(pka, oalpha.at[cid, pl.ds(rowi, 1)])

        pltpu.sync_copy(zb, msg_acc.at[objl.at[0]], add=True)
        return 0

    lax.fori_loop(0, _CPW, _chunk, 0)
    plsc.subcore_barrier()

    r0 = sid * _ROWS_PER_TILE
    pltpu.sync_copy(msg_acc.at[pl.ds(r0, _ROWS_PER_TILE)],
                    omsg.at[cid, pl.ds(r0, _ROWS_PER_TILE)])


def _sc_path_kernel(ix_h, al_h, pp_h, pr_h,
                    opath,
                    ixb, alb, afb, objl, ppb, pb,
                    path_acc,
                    s_pp):
    cid = lax.axis_index("c")
    sid = lax.axis_index("s")

    zero16 = jnp.zeros((16,), jnp.float32)

    def _zrow(i, _):
        for k in range(8):
            pb[i, pl.ds(16 * k, 16)] = zero16
        return 0
    lax.fori_loop(0, _CHUNK, _zrow, 0)
    _zero_acc(sid, pb, pb.at[pl.ds(0, 64)], path_acc, zero16)
    plsc.subcore_barrier()

    lane = lax.iota(jnp.int32, 16)
    trash = jnp.full((16,), _TRASH, jnp.int32)

    def _chunk(ci, _):
        sc = sid * _SPT + (ci >> 3)
        j = jnp.bitwise_and(ci, 7)
        rowi = sc * 8 + j

        @pl.when(j == 0)
        def _ld_ix():
            pltpu.sync_copy(ix_h.at[pl.ds(sc * 32, 32)], ixb)

        subr = ixb.at[j]
        relr = ixb.at[8 + j]
        g_pp = pltpu.async_copy(pp_h.at[subr], ppb, s_pp)
        c_al = pltpu.async_copy(al_h.at[pl.ds(rowi, 1)], alb, s_pp)

        def _og(g, _):
            sl = pl.ds(16 * g, 16)
            objl[0, sl] = _localize_obj(ixb[24 + j, sl], cid, trash)
            return 0
        lax.fori_loop(0, _CHUNK // 16, _og, 0)

        g_pp.wait()
        c_al.wait()
        g_pp2 = pltpu.async_copy(pr_h.at[relr], ppb, s_pp, add=True)

        for g2 in range(4):
            w = alb[0, pl.ds(16 * g2, 16)]
            afb[0, pl.ds(16 * g2, 16)] = plsc.bitcast(w << 16, jnp.float32)
            afb[0, pl.ds(64 + 16 * g2, 16)] = plsc.bitcast(
                jnp.bitwise_and(w, jnp.int32(-65536)), jnp.float32)
        g_pp2.wait()

        def _group(g, _):
            base = g * 16
            avs = afb[0, pl.ds(base, 16)]
            for e in range(16):
                row = base + e
                ae = avs[e]
                for k in range(4):
                    sl = pl.ds(16 * k, 16)
                    v = ppb[row, sl]
                    th = 1.0 - 2.0 / (jnp.exp(2.0 * v) + 1.0)
                    pb[row, sl] = ae * th
            return 0

        lax.fori_loop(0, _CHUNK // 16, _group, 0)

        pltpu.sync_copy(pb, path_acc.at[objl.at[0]], add=True)
        return 0

    lax.fori_loop(0, _CPW, _chunk, 0)
    plsc.subcore_barrier()

    r0 = sid * _ROWS_PER_TILE
    pltpu.sync_copy(path_acc.at[pl.ds(r0, _ROWS_PER_TILE)],
                    opath.at[cid, pl.ds(r0, _ROWS_PER_TILE)])


def kernel(q_sub, q_rel, hidden, path_state, edges, nodes, old_nodes_new_idx,
           batchsize, rela_embed, Ws_attn, Wr_attn, Wqr_attn_w, Wqr_attn_b,
           w_alpha_w, w_alpha_b, W_h, W_path_prev, W_path_rel, curvature):
    f32 = jnp.float32
    i32 = jnp.int32
    n_node = hidden.shape[0]
    c = jnp.maximum(curvature.astype(f32), 1e-6)
    sqrtc = jnp.sqrt(c)
    mxn = (1.0 - 0.004) / sqrtc
    cvec = jnp.zeros((1, 128), f32).at[0, 0].set(c)

    hid_p = jnp.pad(hidden, ((0, _NPAD - hidden.shape[0]), (0, 0)))
    pth_p = jnp.pad(path_state, ((0, _NPAD - path_state.shape[0]), (0, 0)))
    rel_p = jnp.pad(rela_embed, ((0, _NPAD - rela_embed.shape[0]), (0, 0)))
    bq = jnp.broadcast_to(Wqr_attn_b[None, :], (1, 128))

    nblk = _NPAD // 256
    row_spec = lambda d: pl.BlockSpec((256, d), lambda i: (i, 0))
    full = lambda a, b: pl.BlockSpec((a, b), lambda i: (0, 0))
    out128 = jax.ShapeDtypeStruct((_NPAD, 128), f32)
    zs, eh, pp, zr, zq, er, pr = pl.pallas_call(
        _pre_body,
        grid=(nblk,),
        in_specs=[full(1, 128), row_spec(128), row_spec(64), row_spec(128),
                  full(128, 128), full(128, 128), full(128, 128), full(1, 128),
                  full(64, 128), full(128, 128)],
        out_specs=[row_spec(128)] * 7,
        out_shape=[out128] * 7,
    )(cvec, hid_p, pth_p, rel_p, Ws_attn, Wr_attn, Wqr_attn_w, bq,
      jnp.pad(W_path_prev, ((0, 0), (0, 64))),
      jnp.pad(W_path_rel, ((0, 0), (0, 64))))

    epad = _EPAD - edges.shape[0]
    col = lambda j: jnp.pad(edges[:, j].astype(i32), (0, epad)).reshape(_NSC, 8, _CHUNK)
    ix = jnp.stack([col(4), col(2), col(0), col(5)], axis=1)
    ix = ix.reshape(_NSC * 32, _CHUNK)
    qrel2d = jnp.pad(q_rel.astype(i32), (0, _NPAD - q_rel.shape[0])).reshape(80, 128)
    wal_row = lax.bitcast_convert_type(w_alpha_w[:, 0], i32)[None, :]
    cst_row = lax.bitcast_convert_type(
        jnp.repeat(jnp.stack([c, sqrtc, mxn, w_alpha_b[0]]), 16), i32)
    cst_row = jnp.pad(cst_row, (0, 64))[None, :]
    qrel_aux = jnp.concatenate(
        [qrel2d, wal_row, cst_row, jnp.zeros((6, 128), i32)], axis=0)

    mesh = plsc.VectorSubcoreMesh(core_axis_name="c", subcore_axis_name="s")
    cpar = pltpu.CompilerParams(needs_layout_passes=False)
    msg_call = pl.kernel(
        _sc_msg_kernel,
        out_type=[jax.ShapeDtypeStruct((2, _NHALF, 128), f32),
                  jax.ShapeDtypeStruct((2, _NROW, 64), i32)],
        mesh=mesh,
        compiler_params=cpar,
        scratch_types=[
            pltpu.VMEM((88, 128), i32),
            pltpu.VMEM((32, 128), i32),
            pltpu.VMEM((1, _CHUNK), i32),
            pltpu.VMEM((1, _CHUNK), i32),
            pltpu.VMEM((1, _CHUNK), f32),
            pltpu.VMEM((1, 64), i32),
            pltpu.VMEM((_CHUNK, 128), f32),
            pltpu.VMEM((_CHUNK, 128), f32),
            pltpu.VMEM((_CHUNK, 128), f32),
            pltpu.VMEM_SHARED((_NHALF + 8, 128), f32),
            pltpu.SemaphoreType.DMA,
            pltpu.SemaphoreType.DMA,
            pltpu.SemaphoreType.DMA,
            pltpu.SemaphoreType.DMA,
        ],
    )
    omsg, oalpha = msg_call(ix, qrel_aux, zs, zr, zq, eh, er)

    path_call = pl.kernel(
        _sc_path_kernel,
        out_type=[jax.ShapeDtypeStruct((2, _NHALF, 128), f32)],
        mesh=mesh,
        compiler_params=cpar,
        scratch_types=[
            pltpu.VMEM((32, 128), i32),
            pltpu.VMEM((1, 64), i32),
            pltpu.VMEM((1, _CHUNK), f32),
            pltpu.VMEM((1, _CHUNK), i32),
            pltpu.VMEM((_CHUNK, 128), f32),
            pltpu.VMEM((_CHUNK, 128), f32),
            pltpu.VMEM_SHARED((_NHALF + 8, 128), f32),
            pltpu.SemaphoreType.DMA,
        ],
    )
    opath, = path_call(ix, oalpha[0], pp, pr)

    magg = omsg.reshape(_NPAD, 128)
    path_new = opath.reshape(_NPAD, 128)[:n_node, :64]

    hval, = pl.pallas_call(
        _post_body,
        grid=(nblk,),
        in_specs=[full(1, 128), row_spec(128), full(128, 128)],
        out_specs=[row_spec(128)],
        out_shape=[out128],
    )(cvec, magg, W_h)

    return hval[:n_node], path_new


# speedup vs baseline: 1.6734x; 1.0680x over previous
import functools
import numpy as np
import jax
import jax.numpy as jnp
from jax import lax
from jax.experimental import pallas as pl
from jax.experimental.pallas import tpu as pltpu
from jax.experimental.pallas import tpu_sc as plsc

_MIN = 1e-15
_NPAD = 10240
_NHALF = _NPAD // 2
_TRASH = _NHALF
_NEDGE = 320000
_CHUNK = 128
_SPT = 20
_CPW = _SPT * 8
_EPAD = 16 * _CPW * _CHUNK
_NSC = _EPAD // (8 * _CHUNK)
_NROW = _EPAD // _CHUNK
_ROWS_PER_TILE = _NHALF // 16


def _expmap0_rows(u, sqrtc):
    un = jnp.maximum(jnp.sqrt(jnp.sum(u * u, axis=-1, keepdims=True)), _MIN)
    g = jnp.tanh(jnp.clip(sqrtc * un, -15.0, 15.0)) * u / (sqrtc * un)
    gn = jnp.maximum(jnp.sqrt(jnp.sum(g * g, axis=-1, keepdims=True)), _MIN)
    mx = (1.0 - 0.004) / sqrtc
    return jnp.where(gn > mx, g / gn * mx, g)


def _pre_body(cv, h, p, r, ws, wr, wq, bq, wpp, wpr,
              zs, eh, pp, zr, zq, er, pr):
    sqrtc = jnp.sqrt(jnp.maximum(cv[0, 0], 1e-6))
    hv = h[...]
    rv = r[...]
    zs[...] = jnp.dot(hv, ws[...], preferred_element_type=jnp.float32)
    zr[...] = jnp.dot(rv, wr[...], preferred_element_type=jnp.float32)
    zq[...] = jnp.dot(rv, wq[...], preferred_element_type=jnp.float32) + bq[...]
    eh[...] = _expmap0_rows(hv, sqrtc)
    er[...] = _expmap0_rows(rv, sqrtc)
    pp[...] = jnp.dot(p[...], wpp[...], preferred_element_type=jnp.float32)
    pr[...] = jnp.dot(rv, wpr[...], preferred_element_type=jnp.float32)


def _post_body(cv, m, wh, hout):
    sqrtc = jnp.sqrt(jnp.maximum(cv[0, 0], 1e-6))
    a = jnp.dot(m[...], wh[...], preferred_element_type=jnp.float32)
    a = _expmap0_rows(a, sqrtc)
    an = jnp.maximum(jnp.sqrt(jnp.sum(a * a, axis=-1, keepdims=True)), _MIN)
    t = jnp.clip(sqrtc * an, -1.0 + 1e-05, 1.0 - 1e-05)
    art = 0.5 * (jnp.log1p(t) - jnp.log1p(-t))
    hout[...] = a / an / sqrtc * art


def _vsqrt(n2):
    i = plsc.bitcast(n2, jnp.int32)
    y = plsc.bitcast(np.int32(0x5F3759DF) - (i >> 1), jnp.float32)
    for _ in range(3):
        y = y * (1.5 - 0.5 * n2 * y * y)
    return n2 * y


def _vln(q):
    bits = plsc.bitcast(q, jnp.int32)
    e = ((bits >> 23) & 0xFF) - 127
    m = plsc.bitcast((bits & 0x7FFFFF) | np.int32(0x3F800000), jnp.float32)
    r = (m - 1.0) / (m + 1.0)
    r2 = r * r
    lnm = 2.0 * r * (1.0 + r2 * (1.0 / 3.0 + r2 * (0.2 + r2 * (1.0 / 7.0 + r2 / 9.0))))
    return e.astype(jnp.float32) * np.float32(np.log(2.0)) + lnm


def _bf16_pair(lo, hi):
    bl = plsc.bitcast(lo, jnp.int32)
    bh = plsc.bitcast(hi, jnp.int32)
    rl = lax.shift_right_logical(bl + 0x7FFF + ((bl >> 16) & 1), 16) & 0xFFFF
    rh = lax.shift_right_logical(bh + 0x7FFF + ((bh >> 16) & 1), 16) & 0xFFFF
    return rl | (rh << 16)


def _zero_acc(sid, buf128, buf64, acc, zero16):
    r0 = sid * _ROWS_PER_TILE
    pltpu.sync_copy(buf128, acc.at[pl.ds(r0, 128)])
    pltpu.sync_copy(buf128, acc.at[pl.ds(r0 + 128, 128)])
    pltpu.sync_copy(buf64, acc.at[pl.ds(r0 + 256, 64)])


def _localize_obj(ov, cid, trash):
    loc = ov - cid * _NHALF
    ok = jnp.logical_and(loc >= 0, loc < _NHALF)
    return jnp.where(ok, loc, trash)


def _sc_msg_kernel(ix_h, qrel_h, zs_h, zr_h, zq_h, eh_h, er_h,
                   omsg, oalpha,
                   qrel_t, ixb, qidxb, objl, afb, pka, dxy, dx2, dy2,
                   zb, ehb, erb,
                   msg_acc,
                   s_z, s_eh, s_er, s_q):
    cid = lax.axis_index("c")
    sid = lax.axis_index("s")

    pltpu.sync_copy(qrel_h, qrel_t)

    zero16 = jnp.zeros((16,), jnp.float32)

    def _zrow(i, _):
        for k in range(8):
            zb[i, pl.ds(16 * k, 16)] = zero16
        return 0
    lax.fori_loop(0, _CHUNK, _zrow, 0)
    _zero_acc(sid, zb, zb.at[pl.ds(0, 64)], msg_acc, zero16)
    plsc.subcore_barrier()

    lane = lax.iota(jnp.int32, 16)
    trash = jnp.full((16,), _TRASH, jnp.int32)

    def _chunk(ci, _):
        sc = sid * _SPT + (ci >> 3)
        j = jnp.bitwise_and(ci, 7)
        rowi = sc * 8 + j
        be = rowi * _CHUNK

        @pl.when(j == 0)
        def _ld_ix():
            pltpu.sync_copy(ix_h.at[pl.ds(sc * 32, 32)], ixb)

        subr = ixb.at[j]
        relr = ixb.at[8 + j]
        g_z = pltpu.async_copy(zs_h.at[subr], zb, s_z)
        g_eh = pltpu.async_copy(eh_h.at[subr], ehb, s_eh)
        g_er = pltpu.async_copy(er_h.at[relr], erb, s_er)

        def _qg(g, _):
            sl = pl.ds(16 * g, 16)
            rv = ixb[16 + j, sl]
            hi = lax.shift_right_logical(rv, 7)
            lo = jnp.bitwise_and(rv, 127)
            qidxb[0, sl] = plsc.load_gather(qrel_t, [hi, lo])
            objl[0, sl] = _localize_obj(ixb[24 + j, sl], cid, trash)
            return 0
        lax.fori_loop(0, _CHUNK // 16, _qg, 0)

        g_z.wait()
        g_z2 = pltpu.async_copy(zr_h.at[relr], zb, s_z, add=True)
        g_z3 = pltpu.async_copy(zq_h.at[qidxb.at[0]], zb, s_q, add=True)
        g_eh.wait(); g_er.wait()

        def _dots(g, _):
            base = g * 16
            xyv = jnp.zeros((16,), jnp.float32)
            x2v = jnp.zeros((16,), jnp.float32)
            y2v = jnp.zeros((16,), jnp.float32)
            for e in range(16):
                row = base + e
                axy = ax2 = ay2 = None
                for k in range(8):
                    sl = pl.ds(16 * k, 16)
                    xk = ehb[row, sl]
                    yk = erb[row, sl]
                    if k == 0:
                        axy, ax2, ay2 = xk * yk, xk * xk, yk * yk
                    else:
                        axy = axy + xk * yk
                        ax2 = ax2 + xk * xk
                        ay2 = ay2 + yk * yk
                msk = lane == e
                xyv = jnp.where(msk, jnp.sum(axy), xyv)
                x2v = jnp.where(msk, jnp.sum(ax2), x2v)
                y2v = jnp.where(msk, jnp.sum(ay2), y2v)
            dxy[0, pl.ds(base, 16)] = xyv
            dx2[0, pl.ds(base, 16)] = x2v
            dy2[0, pl.ds(base, 16)] = y2v
            return 0
        lax.fori_loop(0, _CHUNK // 16, _dots, 0)
        g_z2.wait(); g_z3.wait()

        cc = plsc.bitcast(qrel_t[81, pl.ds(0, 16)], jnp.float32)
        scv = plsc.bitcast(qrel_t[81, pl.ds(16, 16)], jnp.float32)
        mxv = plsc.bitcast(qrel_t[81, pl.ds(32, 16)], jnp.float32)
        wbv = plsc.bitcast(qrel_t[81, pl.ds(48, 16)], jnp.float32)
        wvecs = [plsc.bitcast(qrel_t[80, pl.ds(16 * k, 16)], jnp.float32)
                 for k in range(8)]

        def _group(g, _):
            base = g * 16
            zdv = jnp.zeros((16,), jnp.float32)
            for e in range(16):
                row = base + e
                az = None
                for k in range(8):
                    sl = pl.ds(16 * k, 16)
                    zk = jnp.maximum(zb[row, sl], 0.0) * wvecs[k]
                    az = zk if k == 0 else az + zk
                msk = lane == e
                zdv = jnp.where(msk, jnp.sum(az), zdv)
            xyv = dxy[0, pl.ds(base, 16)]
            x2v = dx2[0, pl.ds(base, 16)]
            y2v = dy2[0, pl.ds(base, 16)]

            alpha = 1.0 / (1.0 + jnp.exp(-(zdv + wbv)))
            den = jnp.maximum(1.0 + 2.0 * cc * xyv + cc * cc * x2v * y2v, _MIN)
            av = (1.0 + 2.0 * cc * xyv + cc * y2v) / den
            bv = (1.0 - cc * x2v) / den
            n2 = jnp.maximum(av * av * x2v + 2.0 * av * bv * xyv + bv * bv * y2v, 0.0)
            n = _vsqrt(n2)
            ncl = jnp.maximum(n, _MIN)
            scale1 = jnp.where(ncl > mxv, mxv / ncl, 1.0)
            yn = jnp.maximum(n * scale1, _MIN)
            t = scv * yn
            art = 0.5 * _vln((1.0 + t) / (1.0 - t))
            kms = scale1 * art / t
            eidv = lane + (be + base)
            vf = jnp.where(eidv < _NEDGE, 1.0, 0.0)
            akm = alpha * kms * vf
            w1v = akm * av
            w2v = akm * bv
            afb[0, pl.ds(base, 16)] = alpha * vf

            for e in range(16):
                row = base + e
                w1e = w1v[e]
                w2e = w2v[e]
                for k in range(8):
                    sl = pl.ds(16 * k, 16)
                    zb[row, sl] = w1e * ehb[row, sl] + w2e * erb[row, sl]
            return 0

        lax.fori_loop(0, _CHUNK // 16, _group, 0)

        for g in range(4):
            sl = pl.ds(16 * g, 16)
            pka[0, sl] = _bf16_pair(afb[0, sl], afb[0, pl.ds(64 + 16 * g, 16)])
        pltpu.sync_copy(pka, oalpha.at[cid, pl.ds(rowi, 1)])

        pltpu.sync_copy(zb, msg_acc.at[objl.at[0]], add=True)
        return 0

    lax.fori_loop(0, _CPW, _chunk, 0)
    plsc.subcore_barrier()

    r0 = sid * _ROWS_PER_TILE
    pltpu.sync_copy(msg_acc.at[pl.ds(r0, _ROWS_PER_TILE)],
                    omsg.at[cid, pl.ds(r0, _ROWS_PER_TILE)])


def _sc_path_kernel(ix_h, al_h, pp_h, pr_h,
                    opath,
                    ixb, alb, afb, objl, ppb, pb,
                    path_acc,
                    s_pp):
    cid = lax.axis_index("c")
    sid = lax.axis_index("s")

    zero16 = jnp.zeros((16,), jnp.float32)

    def _zrow(i, _):
        for k in range(8):
            pb[i, pl.ds(16 * k, 16)] = zero16
        return 0
    lax.fori_loop(0, _CHUNK, _zrow, 0)
    _zero_acc(sid, pb, pb.at[pl.ds(0, 64)], path_acc, zero16)
    plsc.subcore_barrier()

    lane = lax.iota(jnp.int32, 16)
    trash = jnp.full((16,), _TRASH, jnp.int32)

    def _chunk(ci, _):
        sc = sid * _SPT + (ci >> 3)
        j = jnp.bitwise_and(ci, 7)
        rowi = sc * 8 + j

        @pl.when(j == 0)
        def _ld_ix():
            pltpu.sync_copy(ix_h.at[pl.ds(sc * 32, 32)], ixb)

        subr = ixb.at[j]
        relr = ixb.at[8 + j]
        g_pp = pltpu.async_copy(pp_h.at[subr], ppb, s_pp)
        c_al = pltpu.async_copy(al_h.at[pl.ds(rowi, 1)], alb, s_pp)

        def _og(g, _):
            sl = pl.ds(16 * g, 16)
            objl[0, sl] = _localize_obj(ixb[24 + j, sl], cid, trash)
            return 0
        lax.fori_loop(0, _CHUNK // 16, _og, 0)

        g_pp.wait()
        c_al.wait()
        g_pp2 = pltpu.async_copy(pr_h.at[relr], ppb, s_pp, add=True)

        for g2 in range(4):
            w = alb[0, pl.ds(16 * g2, 16)]
            afb[0, pl.ds(16 * g2, 16)] = plsc.bitcast(w << 16, jnp.float32)
            afb[0, pl.ds(64 + 16 * g2, 16)] = plsc.bitcast(
                jnp.bitwise_and(w, jnp.int32(-65536)), jnp.float32)
        g_pp2.wait()

        def _group(g, _):
            base = g * 16
            avs = afb[0, pl.ds(base, 16)]
            for e in range(16):
                row = base + e
                ae = avs[e]
                for k in range(4):
                    sl = pl.ds(16 * k, 16)
                    v = ppb[row, sl]
                    th = 1.0 - 2.0 / (jnp.exp(2.0 * v) + 1.0)
                    pb[row, sl] = ae * th
            return 0

        lax.fori_loop(0, _CHUNK // 16, _group, 0)

        pltpu.sync_copy(pb, path_acc.at[objl.at[0]], add=True)
        return 0

    lax.fori_loop(0, _CPW, _chunk, 0)
    plsc.subcore_barrier()

    r0 = sid * _ROWS_PER_TILE
    pltpu.sync_copy(path_acc.at[pl.ds(r0, _ROWS_PER_TILE)],
                    opath.at[cid, pl.ds(r0, _ROWS_PER_TILE)])


def kernel(q_sub, q_rel, hidden, path_state, edges, nodes, old_nodes_new_idx,
           batchsize, rela_embed, Ws_attn, Wr_attn, Wqr_attn_w, Wqr_attn_b,
           w_alpha_w, w_alpha_b, W_h, W_path_prev, W_path_rel, curvature):
    f32 = jnp.float32
    i32 = jnp.int32
    n_node = hidden.shape[0]
    c = jnp.maximum(curvature.astype(f32), 1e-6)
    sqrtc = jnp.sqrt(c)
    mxn = (1.0 - 0.004) / sqrtc
    cvec = jnp.zeros((1, 128), f32).at[0, 0].set(c)

    hid_p = jnp.pad(hidden, ((0, _NPAD - hidden.shape[0]), (0, 0)))
    pth_p = jnp.pad(path_state, ((0, _NPAD - path_state.shape[0]), (0, 0)))
    rel_p = jnp.pad(rela_embed, ((0, _NPAD - rela_embed.shape[0]), (0, 0)))
    bq = jnp.broadcast_to(Wqr_attn_b[None, :], (1, 128))

    nblk = _NPAD // 256
    row_spec = lambda d: pl.BlockSpec((256, d), lambda i: (i, 0))
    full = lambda a, b: pl.BlockSpec((a, b), lambda i: (0, 0))
    out128 = jax.ShapeDtypeStruct((_NPAD, 128), f32)
    zs, eh, pp, zr, zq, er, pr = pl.pallas_call(
        _pre_body,
        grid=(nblk,),
        in_specs=[full(1, 128), row_spec(128), row_spec(64), row_spec(128),
                  full(128, 128), full(128, 128), full(128, 128), full(1, 128),
                  full(64, 128), full(128, 128)],
        out_specs=[row_spec(128)] * 7,
        out_shape=[out128] * 7,
    )(cvec, hid_p, pth_p, rel_p, Ws_attn, Wr_attn, Wqr_attn_w, bq,
      jnp.pad(W_path_prev, ((0, 0), (0, 64))),
      jnp.pad(W_path_rel, ((0, 0), (0, 64))))

    epad = _EPAD - edges.shape[0]
    col = lambda j: jnp.pad(edges[:, j].astype(i32), (0, epad)).reshape(_NSC, 8, _CHUNK)
    ix = jnp.stack([col(4), col(2), col(0), col(5)], axis=1)
    ix = ix.reshape(_NSC * 32, _CHUNK)
    qrel2d = jnp.pad(q_rel.astype(i32), (0, _NPAD - q_rel.shape[0])).reshape(80, 128)
    wal_row = lax.bitcast_convert_type(w_alpha_w[:, 0], i32)[None, :]
    cst_row = lax.bitcast_convert_type(
        jnp.repeat(jnp.stack([c, sqrtc, mxn, w_alpha_b[0]]), 16), i32)
    cst_row = jnp.pad(cst_row, (0, 64))[None, :]
    qrel_aux = jnp.concatenate(
        [qrel2d, wal_row, cst_row, jnp.zeros((6, 128), i32)], axis=0)

    mesh = plsc.VectorSubcoreMesh(core_axis_name="c", subcore_axis_name="s")
    cpar = pltpu.CompilerParams(needs_layout_passes=False)
    msg_call = pl.kernel(
        _sc_msg_kernel,
        out_type=[jax.ShapeDtypeStruct((2, _NHALF, 128), f32),
                  jax.ShapeDtypeStruct((2, _NROW, 64), i32)],
        mesh=mesh,
        compiler_params=cpar,
        scratch_types=[
            pltpu.VMEM((88, 128), i32),
            pltpu.VMEM((32, 128), i32),
            pltpu.VMEM((1, _CHUNK), i32),
            pltpu.VMEM((1, _CHUNK), i32),
            pltpu.VMEM((1, _CHUNK), f32),
            pltpu.VMEM((1, 64), i32),
            pltpu.VMEM((1, _CHUNK), f32),
            pltpu.VMEM((1, _CHUNK), f32),
            pltpu.VMEM((1, _CHUNK), f32),
            pltpu.VMEM((_CHUNK, 128), f32),
            pltpu.VMEM((_CHUNK, 128), f32),
            pltpu.VMEM((_CHUNK, 128), f32),
            pltpu.VMEM_SHARED((_NHALF + 8, 128), f32),
            pltpu.SemaphoreType.DMA,
            pltpu.SemaphoreType.DMA,
            pltpu.SemaphoreType.DMA,
            pltpu.SemaphoreType.DMA,
        ],
    )
    omsg, oalpha = msg_call(ix, qrel_aux, zs, zr, zq, eh, er)

    path_call = pl.kernel(
        _sc_path_kernel,
        out_type=[jax.ShapeDtypeStruct((2, _NHALF, 128), f32)],
        mesh=mesh,
        compiler_params=cpar,
        scratch_types=[
            pltpu.VMEM((32, 128), i32),
            pltpu.VMEM((1, 64), i32),
            pltpu.VMEM((1, _CHUNK), f32),
            pltpu.VMEM((1, _CHUNK), i32),
            pltpu.VMEM((_CHUNK, 128), f32),
            pltpu.VMEM((_CHUNK, 128), f32),
            pltpu.VMEM_SHARED((_NHALF + 8, 128), f32),
            pltpu.SemaphoreType.DMA,
        ],
    )
    opath, = path_call(ix, oalpha[0], pp, pr)

    magg = omsg.reshape(_NPAD, 128)
    path_new = opath.reshape(_NPAD, 128)[:n_node, :64]

    hval, = pl.pallas_call(
        _post_body,
        grid=(nblk,),
        in_specs=[full(1, 128), row_spec(128), full(128, 128)],
        out_specs=[row_spec(128)],
        out_shape=[out128],
    )(cvec, magg, W_h)

    return hval[:n_node], path_new


# speedup vs baseline: 1.7251x; 1.0308x over previous
import functools
import numpy as np
import jax
import jax.numpy as jnp
from jax import lax
from jax.experimental import pallas as pl
from jax.experimental.pallas import tpu as pltpu
from jax.experimental.pallas import tpu_sc as plsc

_MIN = 1e-15
_NPAD = 10240
_NHALF = _NPAD // 2
_TRASH = _NHALF
_NEDGE = 320000
_CHUNK = 128
_SPT = 20
_CPW = _SPT * 8
_EPAD = 16 * _CPW * _CHUNK
_NSC = _EPAD // (8 * _CHUNK)
_NROW = _EPAD // _CHUNK
_ROWS_PER_TILE = _NHALF // 16


def _expmap0_rows(u, sqrtc):
    un = jnp.maximum(jnp.sqrt(jnp.sum(u * u, axis=-1, keepdims=True)), _MIN)
    g = jnp.tanh(jnp.clip(sqrtc * un, -15.0, 15.0)) * u / (sqrtc * un)
    gn = jnp.maximum(jnp.sqrt(jnp.sum(g * g, axis=-1, keepdims=True)), _MIN)
    mx = (1.0 - 0.004) / sqrtc
    return jnp.where(gn > mx, g / gn * mx, g)


def _pre_body(cv, h, p, r, ws, wr, wq, bq, wpp, wpr,
              zs, eh, pp, zr, zq, er, pr):
    sqrtc = jnp.sqrt(jnp.maximum(cv[0, 0], 1e-6))
    hv = h[...]
    rv = r[...]
    zs[...] = jnp.dot(hv, ws[...], preferred_element_type=jnp.float32)
    zr[...] = jnp.dot(rv, wr[...], preferred_element_type=jnp.float32)
    zq[...] = jnp.dot(rv, wq[...], preferred_element_type=jnp.float32) + bq[...]
    eh[...] = _expmap0_rows(hv, sqrtc)
    er[...] = _expmap0_rows(rv, sqrtc)
    pp[...] = jnp.dot(p[...], wpp[...], preferred_element_type=jnp.float32)
    pr[...] = jnp.dot(rv, wpr[...], preferred_element_type=jnp.float32)


def _post_body(cv, m, wh, hout):
    sqrtc = jnp.sqrt(jnp.maximum(cv[0, 0], 1e-6))
    a = jnp.dot(m[...], wh[...], preferred_element_type=jnp.float32)
    a = _expmap0_rows(a, sqrtc)
    an = jnp.maximum(jnp.sqrt(jnp.sum(a * a, axis=-1, keepdims=True)), _MIN)
    t = jnp.clip(sqrtc * an, -1.0 + 1e-05, 1.0 - 1e-05)
    art = 0.5 * (jnp.log1p(t) - jnp.log1p(-t))
    hout[...] = a / an / sqrtc * art


def _vsqrt(n2):
    i = plsc.bitcast(n2, jnp.int32)
    y = plsc.bitcast(np.int32(0x5F3759DF) - (i >> 1), jnp.float32)
    for _ in range(3):
        y = y * (1.5 - 0.5 * n2 * y * y)
    return n2 * y


def _vln(q):
    bits = plsc.bitcast(q, jnp.int32)
    e = ((bits >> 23) & 0xFF) - 127
    m = plsc.bitcast((bits & 0x7FFFFF) | np.int32(0x3F800000), jnp.float32)
    r = (m - 1.0) / (m + 1.0)
    r2 = r * r
    lnm = 2.0 * r * (1.0 + r2 * (1.0 / 3.0 + r2 * (0.2 + r2 * (1.0 / 7.0 + r2 / 9.0))))
    return e.astype(jnp.float32) * np.float32(np.log(2.0)) + lnm


def _bf16_pair(lo, hi):
    bl = plsc.bitcast(lo, jnp.int32)
    bh = plsc.bitcast(hi, jnp.int32)
    rl = lax.shift_right_logical(bl + 0x7FFF + ((bl >> 16) & 1), 16) & 0xFFFF
    rh = lax.shift_right_logical(bh + 0x7FFF + ((bh >> 16) & 1), 16) & 0xFFFF
    return rl | (rh << 16)


def _zero_acc(sid, buf128, buf64, acc, zero16):
    r0 = sid * _ROWS_PER_TILE
    pltpu.sync_copy(buf128, acc.at[pl.ds(r0, 128)])
    pltpu.sync_copy(buf128, acc.at[pl.ds(r0 + 128, 128)])
    pltpu.sync_copy(buf64, acc.at[pl.ds(r0 + 256, 64)])


def _localize_obj(ov, cid, trash):
    loc = ov - cid * _NHALF
    ok = jnp.logical_and(loc >= 0, loc < _NHALF)
    return jnp.where(ok, loc, trash)


def _sc_msg_kernel(ix_h, qrel_h, zs_h, zr_h, zq_h, eh_h, er_h,
                   omsg, oalpha,
                   qrel_t, ixb, qidxb, objl, afb, pka, dxy, dx2, dy2,
                   zb, ehb, erb,
                   msg_acc,
                   s_z, s_eh, s_er, s_q):
    cid = lax.axis_index("c")
    sid = lax.axis_index("s")

    pltpu.sync_copy(qrel_h, qrel_t)

    zero16 = jnp.zeros((16,), jnp.float32)

    def _zrow(i, _):
        for k in range(8):
            zb[i, pl.ds(16 * k, 16)] = zero16
        return 0
    lax.fori_loop(0, _CHUNK, _zrow, 0)
    _zero_acc(sid, zb, zb.at[pl.ds(0, 64)], msg_acc, zero16)
    plsc.subcore_barrier()

    lane = lax.iota(jnp.int32, 16)
    trash = jnp.full((16,), _TRASH, jnp.int32)

    def _chunk(ci, _):
        sc = sid * _SPT + (ci >> 3)
        j = jnp.bitwise_and(ci, 7)
        rowi = sc * 8 + j
        be = rowi * _CHUNK

        @pl.when(j == 0)
        def _ld_ix():
            pltpu.sync_copy(ix_h.at[pl.ds(sc * 32, 32)], ixb)

        subr = ixb.at[j]
        relr = ixb.at[8 + j]
        g_z = pltpu.async_copy(zs_h.at[subr], zb, s_z)
        g_eh = pltpu.async_copy(eh_h.at[subr], ehb, s_eh)
        g_er = pltpu.async_copy(er_h.at[relr], erb, s_er)

        def _qg(g, _):
            sl = pl.ds(16 * g, 16)
            rv = ixb[16 + j, sl]
            hi = lax.shift_right_logical(rv, 7)
            lo = jnp.bitwise_and(rv, 127)
            qidxb[0, sl] = plsc.load_gather(qrel_t, [hi, lo])
            objl[0, sl] = _localize_obj(ixb[24 + j, sl], cid, trash)
            return 0
        lax.fori_loop(0, _CHUNK // 16, _qg, 0)

        g_z.wait()
        g_z2 = pltpu.async_copy(zr_h.at[relr], zb, s_z, add=True)
        g_z3 = pltpu.async_copy(zq_h.at[qidxb.at[0]], zb, s_q, add=True)
        g_eh.wait(); g_er.wait()

        def _dots(g, _):
            base = g * 16
            xyv = jnp.zeros((16,), jnp.float32)
            x2v = jnp.zeros((16,), jnp.float32)
            y2v = jnp.zeros((16,), jnp.float32)
            for e in range(16):
                row = base + e
                axy = ax2 = ay2 = None
                for k in range(8):
                    sl = pl.ds(16 * k, 16)
                    xk = ehb[row, sl]
                    yk = erb[row, sl]
                    if k == 0:
                        axy, ax2, ay2 = xk * yk, xk * xk, yk * yk
                    else:
                        axy = axy + xk * yk
                        ax2 = ax2 + xk * xk
                        ay2 = ay2 + yk * yk
                msk = lane == e
                xyv = jnp.where(msk, jnp.sum(axy), xyv)
                x2v = jnp.where(msk, jnp.sum(ax2), x2v)
                y2v = jnp.where(msk, jnp.sum(ay2), y2v)
            dxy[0, pl.ds(base, 16)] = xyv
            dx2[0, pl.ds(base, 16)] = x2v
            dy2[0, pl.ds(base, 16)] = y2v
            return 0
        lax.fori_loop(0, _CHUNK // 16, _dots, 0)
        g_z2.wait(); g_z3.wait()

        cc = plsc.bitcast(qrel_t[81, pl.ds(0, 16)], jnp.float32)
        scv = plsc.bitcast(qrel_t[81, pl.ds(16, 16)], jnp.float32)
        mxv = plsc.bitcast(qrel_t[81, pl.ds(32, 16)], jnp.float32)
        wbv = plsc.bitcast(qrel_t[81, pl.ds(48, 16)], jnp.float32)
        wvecs = [plsc.bitcast(qrel_t[80, pl.ds(16 * k, 16)], jnp.float32)
                 for k in range(8)]

        def _group(g, _):
            base = g * 16
            zdv = jnp.zeros((16,), jnp.float32)
            for e in range(16):
                row = base + e
                az = None
                for k in range(8):
                    sl = pl.ds(16 * k, 16)
                    zk = jnp.maximum(zb[row, sl], 0.0) * wvecs[k]
                    az = zk if k == 0 else az + zk
                msk = lane == e
                zdv = jnp.where(msk, jnp.sum(az), zdv)
            xyv = dxy[0, pl.ds(base, 16)]
            x2v = dx2[0, pl.ds(base, 16)]
            y2v = dy2[0, pl.ds(base, 16)]

            alpha = 1.0 / (1.0 + jnp.exp(-(zdv + wbv)))
            den = jnp.maximum(1.0 + 2.0 * cc * xyv + cc * cc * x2v * y2v, _MIN)
            av = (1.0 + 2.0 * cc * xyv + cc * y2v) / den
            bv = (1.0 - cc * x2v) / den
            n2 = jnp.maximum(av * av * x2v + 2.0 * av * bv * xyv + bv * bv * y2v, 0.0)
            n = _vsqrt(n2)
            ncl = jnp.maximum(n, _MIN)
            scale1 = jnp.where(ncl > mxv, mxv / ncl, 1.0)
            yn = jnp.maximum(n * scale1, _MIN)
            t = scv * yn
            art = 0.5 * _vln((1.0 + t) / (1.0 - t))
            kms = scale1 * art / t
            eidv = lane + (be + base)
            vf = jnp.where(eidv < _NEDGE, 1.0, 0.0)
            akm = alpha * kms * vf
            w1v = akm * av
            w2v = akm * bv
            afb[0, pl.ds(base, 16)] = alpha * vf

            for e in range(16):
                row = base + e
                w1e = w1v[e]
                w2e = w2v[e]
                for k in range(8):
                    sl = pl.ds(16 * k, 16)
                    zb[row, sl] = w1e * ehb[row, sl] + w2e * erb[row, sl]
            return 0

        lax.fori_loop(0, _CHUNK // 16, _group, 0)

        for g in range(4):
            sl = pl.ds(16 * g, 16)
            pka[0, sl] = _bf16_pair(afb[0, sl], afb[0, pl.ds(64 + 16 * g, 16)])
        pltpu.sync_copy(pka, oalpha.at[cid, pl.ds(rowi, 1)])

        pltpu.sync_copy(zb, msg_acc.at[objl.at[0]], add=True)
        return 0

    lax.fori_loop(0, _CPW, _chunk, 0)
    plsc.subcore_barrier()

    r0 = sid * _ROWS_PER_TILE
    pltpu.sync_copy(msg_acc.at[pl.ds(r0, _ROWS_PER_TILE)],
                    omsg.at[cid, pl.ds(r0, _ROWS_PER_TILE)])


def _sc_path_kernel(ix_h, al_h, pp_h, pr_h,
                    opath,
                    ixb, alb, afb, objl, ppb0, ppb1, pb,
                    path_acc,
                    s_p0, s_p1, s_al):
    cid = lax.axis_index("c")
    sid = lax.axis_index("s")

    zero16 = jnp.zeros((16,), jnp.float32)

    def _zrow(i, _):
        for k in range(8):
            pb[i, pl.ds(16 * k, 16)] = zero16
        return 0
    lax.fori_loop(0, _CHUNK, _zrow, 0)
    _zero_acc(sid, pb, pb.at[pl.ds(0, 64)], path_acc, zero16)
    plsc.subcore_barrier()

    lane = lax.iota(jnp.int32, 16)
    trash = jnp.full((16,), _TRASH, jnp.int32)
    ppbs = (ppb0, ppb1)
    sems = (s_p0, s_p1)

    def _ix_slot_refs(ci):
        sc = sid * _SPT + (ci >> 3)
        j = jnp.bitwise_and(ci, 7)
        slot = jnp.bitwise_and(sc, 1) * 32
        return sc, j, slot

    def _issue_base(ci, buf, sem):
        sc, j, slot = _ix_slot_refs(ci)

        @pl.when(j == 0)
        def _ld_ix():
            pltpu.sync_copy(ix_h.at[pl.ds(sc * 32, 32)],
                            ixb.at[pl.ds(slot, 32)])
        pltpu.async_copy(pp_h.at[ixb.at[slot + j]], buf, sem)

    _issue_base(0, ppb0, s_p0)

    def _pair(p, _):
        for b in (0, 1):
            ci = 2 * p + b
            sc, j, slot = _ix_slot_refs(ci)
            rowi = sc * 8 + j
            buf = ppbs[b]
            sem = sems[b]
            c_al = pltpu.async_copy(al_h.at[pl.ds(rowi, 1)], alb, s_al)

            def _og(g, _):
                sl = pl.ds(16 * g, 16)
                objl[0, sl] = _localize_obj(ixb[slot + 24 + j, sl], cid, trash)
                return 0
            lax.fori_loop(0, _CHUNK // 16, _og, 0)

            pltpu.make_async_copy(pp_h.at[ixb.at[slot + j]], buf, sem).wait()
            pltpu.async_copy(pr_h.at[ixb.at[slot + 8 + j]], buf, sem,
                             add=True)

            @pl.when(ci + 1 < _CPW)
            def _pre():
                _issue_base(ci + 1, ppbs[1 - b], sems[1 - b])

            c_al.wait()
            for g2 in range(4):
                w = alb[0, pl.ds(16 * g2, 16)]
                afb[0, pl.ds(16 * g2, 16)] = plsc.bitcast(w << 16, jnp.float32)
                afb[0, pl.ds(64 + 16 * g2, 16)] = plsc.bitcast(
                    jnp.bitwise_and(w, jnp.int32(-65536)), jnp.float32)
            pltpu.make_async_copy(pr_h.at[ixb.at[slot + 8 + j]], buf,
                                  sem).wait()

            def _group(g, _):
                base = g * 16
                avs = afb[0, pl.ds(base, 16)]
                for e in range(16):
                    row = base + e
                    ae = avs[e]
                    for k in range(4):
                        sl = pl.ds(16 * k, 16)
                        v = buf[row, sl]
                        th = 1.0 - 2.0 / (jnp.exp(2.0 * v) + 1.0)
                        pb[row, sl] = ae * th
                return 0

            lax.fori_loop(0, _CHUNK // 16, _group, 0)

            pltpu.sync_copy(pb, path_acc.at[objl.at[0]], add=True)
        return 0

    lax.fori_loop(0, _CPW // 2, _pair, 0)
    plsc.subcore_barrier()

    r0 = sid * _ROWS_PER_TILE
    pltpu.sync_copy(path_acc.at[pl.ds(r0, _ROWS_PER_TILE)],
                    opath.at[cid, pl.ds(r0, _ROWS_PER_TILE)])


def kernel(q_sub, q_rel, hidden, path_state, edges, nodes, old_nodes_new_idx,
           batchsize, rela_embed, Ws_attn, Wr_attn, Wqr_attn_w, Wqr_attn_b,
           w_alpha_w, w_alpha_b, W_h, W_path_prev, W_path_rel, curvature):
    f32 = jnp.float32
    i32 = jnp.int32
    n_node = hidden.shape[0]
    c = jnp.maximum(curvature.astype(f32), 1e-6)
    sqrtc = jnp.sqrt(c)
    mxn = (1.0 - 0.004) / sqrtc
    cvec = jnp.zeros((1, 128), f32).at[0, 0].set(c)

    hid_p = jnp.pad(hidden, ((0, _NPAD - hidden.shape[0]), (0, 0)))
    pth_p = jnp.pad(path_state, ((0, _NPAD - path_state.shape[0]), (0, 0)))
    rel_p = jnp.pad(rela_embed, ((0, _NPAD - rela_embed.shape[0]), (0, 0)))
    bq = jnp.broadcast_to(Wqr_attn_b[None, :], (1, 128))

    nblk = _NPAD // 256
    row_spec = lambda d: pl.BlockSpec((256, d), lambda i: (i, 0))
    full = lambda a, b: pl.BlockSpec((a, b), lambda i: (0, 0))
    out128 = jax.ShapeDtypeStruct((_NPAD, 128), f32)
    zs, eh, pp, zr, zq, er, pr = pl.pallas_call(
        _pre_body,
        grid=(nblk,),
        in_specs=[full(1, 128), row_spec(128), row_spec(64), row_spec(128),
                  full(128, 128), full(128, 128), full(128, 128), full(1, 128),
                  full(64, 128), full(128, 128)],
        out_specs=[row_spec(128)] * 7,
        out_shape=[out128] * 7,
    )(cvec, hid_p, pth_p, rel_p, Ws_attn, Wr_attn, Wqr_attn_w, bq,
      jnp.pad(W_path_prev, ((0, 0), (0, 64))),
      jnp.pad(W_path_rel, ((0, 0), (0, 64))))

    epad = _EPAD - edges.shape[0]
    col = lambda j: jnp.pad(edges[:, j].astype(i32), (0, epad)).reshape(_NSC, 8, _CHUNK)
    ix = jnp.stack([col(4), col(2), col(0), col(5)], axis=1)
    ix = ix.reshape(_NSC * 32, _CHUNK)
    qrel2d = jnp.pad(q_rel.astype(i32), (0, _NPAD - q_rel.shape[0])).reshape(80, 128)
    wal_row = lax.bitcast_convert_type(w_alpha_w[:, 0], i32)[None, :]
    cst_row = lax.bitcast_convert_type(
        jnp.repeat(jnp.stack([c, sqrtc, mxn, w_alpha_b[0]]), 16), i32)
    cst_row = jnp.pad(cst_row, (0, 64))[None, :]
    qrel_aux = jnp.concatenate(
        [qrel2d, wal_row, cst_row, jnp.zeros((6, 128), i32)], axis=0)

    mesh = plsc.VectorSubcoreMesh(core_axis_name="c", subcore_axis_name="s")
    cpar = pltpu.CompilerParams(needs_layout_passes=False)
    msg_call = pl.kernel(
        _sc_msg_kernel,
        out_type=[jax.ShapeDtypeStruct((2, _NHALF, 128), f32),
                  jax.ShapeDtypeStruct((2, _NROW, 64), i32)],
        mesh=mesh,
        compiler_params=cpar,
        scratch_types=[
            pltpu.VMEM((88, 128), i32),
            pltpu.VMEM((32, 128), i32),
            pltpu.VMEM((1, _CHUNK), i32),
            pltpu.VMEM((1, _CHUNK), i32),
            pltpu.VMEM((1, _CHUNK), f32),
            pltpu.VMEM((1, 64), i32),
            pltpu.VMEM((1, _CHUNK), f32),
            pltpu.VMEM((1, _CHUNK), f32),
            pltpu.VMEM((1, _CHUNK), f32),
            pltpu.VMEM((_CHUNK, 128), f32),
            pltpu.VMEM((_CHUNK, 128), f32),
            pltpu.VMEM((_CHUNK, 128), f32),
            pltpu.VMEM_SHARED((_NHALF + 8, 128), f32),
            pltpu.SemaphoreType.DMA,
            pltpu.SemaphoreType.DMA,
            pltpu.SemaphoreType.DMA,
            pltpu.SemaphoreType.DMA,
        ],
    )
    omsg, oalpha = msg_call(ix, qrel_aux, zs, zr, zq, eh, er)

    path_call = pl.kernel(
        _sc_path_kernel,
        out_type=[jax.ShapeDtypeStruct((2, _NHALF, 128), f32)],
        mesh=mesh,
        compiler_params=cpar,
        scratch_types=[
            pltpu.VMEM((64, 128), i32),
            pltpu.VMEM((1, 64), i32),
            pltpu.VMEM((1, _CHUNK), f32),
            pltpu.VMEM((1, _CHUNK), i32),
            pltpu.VMEM((_CHUNK, 128), f32),
            pltpu.VMEM((_CHUNK, 128), f32),
            pltpu.VMEM((_CHUNK, 128), f32),
            pltpu.VMEM_SHARED((_NHALF + 8, 128), f32),
            pltpu.SemaphoreType.DMA,
            pltpu.SemaphoreType.DMA,
            pltpu.SemaphoreType.DMA,
        ],
    )
    opath, = path_call(ix, oalpha[0], pp, pr)

    magg = omsg.reshape(_NPAD, 128)
    path_new = opath.reshape(_NPAD, 128)[:n_node, :64]

    hval, = pl.pallas_call(
        _post_body,
        grid=(nblk,),
        in_specs=[full(1, 128), row_spec(128), full(128, 128)],
        out_specs=[row_spec(128)],
        out_shape=[out128],
    )(cvec, magg, W_h)

    return hval[:n_node], path_new
